# baseline (speedup 1.0000x reference)
import functools

import jax
import jax.numpy as jnp
from jax import lax
from jax.experimental import pallas as pl
from jax.experimental.pallas import tpu as pltpu
from jax.experimental.pallas import tpu_sc as plsc

N = 10000
E = 320000
D = 128
H = 128
EXTRA = 16
HD = D // 2

NC = 2
NS = 16
CH = 128
EPAD = NC * NS * 80 * CH
NROW = EPAD // CH
NPAD = 10240
RPT = NPAD // NS
DEG_ROWS = NROW // (NC * NS)
AGG_ROWS = NROW // NS

_SELU_ALPHA = 1.6732632423543772
_SELU_SCALE = 1.0507009873554805

_MESH = dict(core_axis_name="c", subcore_axis_name="s", num_cores=NC,
             num_subcores=NS)


def _selu(x):
    return _SELU_SCALE * jnp.where(x > 0, x, _SELU_ALPHA * (jnp.exp(x) - 1.0))


@functools.partial(
    pl.kernel,
    out_type=[jax.ShapeDtypeStruct((NPAD,), jnp.float32),
              jax.ShapeDtypeStruct((NPAD,), jnp.float32),
              jax.ShapeDtypeStruct((NPAD,), jnp.float32),
              jax.ShapeDtypeStruct((NPAD,), jnp.float32)],
    mesh=plsc.VectorSubcoreMesh(**_MESH),
    compiler_params=pltpu.CompilerParams(use_tc_tiling_on_sc=False),
    scratch_types=[
        pltpu.VMEM((DEG_ROWS, CH), jnp.int32),
        pltpu.VMEM((DEG_ROWS, CH), jnp.int32),
        pltpu.VMEM((CH,), jnp.float32),
        pltpu.VMEM((RPT,), jnp.float32),
        pltpu.VMEM_SHARED((NPAD,), jnp.float32),
        pltpu.VMEM_SHARED((NPAD,), jnp.float32),
    ],
)
def _deg_kernel(src_h, dst_h, dgo0_h, dgo1_h, dgi0_h, dgi1_h,
                idxs_v, idxd_v, ones_v, zbuf_v, dego_sh, degi_sh):
    cid = lax.axis_index("c")
    sid = lax.axis_index("s")
    base = (cid * NS + sid) * DEG_ROWS

    for i in range(CH // 16):
        ones_v[pl.ds(i * 16, 16)] = jnp.ones((16,), jnp.float32)
    for i in range(RPT // 16):
        zbuf_v[pl.ds(i * 16, 16)] = jnp.zeros((16,), jnp.float32)
    pltpu.sync_copy(zbuf_v, dego_sh.at[pl.ds(sid * RPT, RPT)])
    pltpu.sync_copy(zbuf_v, degi_sh.at[pl.ds(sid * RPT, RPT)])
    pltpu.sync_copy(src_h.at[pl.ds(base, DEG_ROWS)], idxs_v)
    pltpu.sync_copy(dst_h.at[pl.ds(base, DEG_ROWS)], idxd_v)
    plsc.subcore_barrier()

    def body(j, carry):
        pltpu.sync_copy(ones_v, dego_sh.at[idxs_v.at[j]], add=True)
        pltpu.sync_copy(ones_v, degi_sh.at[idxd_v.at[j]], add=True)
        return carry

    lax.fori_loop(0, DEG_ROWS, body, 0)
    plsc.subcore_barrier()

    @pl.when(cid == 0)
    def _():
        pltpu.sync_copy(dego_sh.at[pl.ds(sid * RPT, RPT)],
                        dgo0_h.at[pl.ds(sid * RPT, RPT)])
        pltpu.sync_copy(degi_sh.at[pl.ds(sid * RPT, RPT)],
                        dgi0_h.at[pl.ds(sid * RPT, RPT)])

    @pl.when(cid == 1)
    def _():
        pltpu.sync_copy(dego_sh.at[pl.ds(sid * RPT, RPT)],
                        dgo1_h.at[pl.ds(sid * RPT, RPT)])
        pltpu.sync_copy(degi_sh.at[pl.ds(sid * RPT, RPT)],
                        dgi1_h.at[pl.ds(sid * RPT, RPT)])


NPH = 4
PHR = AGG_ROWS // NPH


@functools.partial(
    pl.kernel,
    out_type=[jax.ShapeDtypeStruct((NPAD, HD), jnp.float32),
              jax.ShapeDtypeStruct((NPAD, HD), jnp.float32)],
    mesh=plsc.VectorSubcoreMesh(**_MESH),
    compiler_params=pltpu.CompilerParams(use_tc_tiling_on_sc=False),
    scratch_types=[
        pltpu.VMEM((PHR, CH), jnp.int32),
        pltpu.VMEM((PHR, CH), jnp.int32),
        pltpu.VMEM((CH, HD), jnp.float32),
        pltpu.VMEM((CH, HD), jnp.float32),
        pltpu.VMEM((CH, HD), jnp.float32),
        pltpu.VMEM((CH, HD), jnp.float32),
        pltpu.VMEM_SHARED((NPAD, HD), jnp.float32),
        pltpu.VMEM_SHARED((NPAD, HD), jnp.float32),
        pltpu.SemaphoreType.DMA,
        pltpu.SemaphoreType.DMA,
        pltpu.SemaphoreType.DMA,
        pltpu.SemaphoreType.DMA,
    ],
)
def _agg_kernel(src_h, dst_h, xa_h, xb_h, zeros_h, pa_h, pb_h,
                idxs_v, idxd_v, r0, r1, r2, r3,
                xs_sh, acc_sh, g0, g1, g2, g3):
    cid = lax.axis_index("c")
    sid = lax.axis_index("s")
    base = sid * AGG_ROWS
    rows = (r0, r1, r2, r3)
    gs = (g0, g1, g2, g3)

    pltpu.sync_copy(zeros_h.at[pl.ds(sid * RPT, RPT)],
                    acc_sh.at[pl.ds(sid * RPT, RPT)])

    @pl.when(cid == 0)
    def _():
        pltpu.sync_copy(xa_h.at[pl.ds(sid * RPT, RPT)],
                        xs_sh.at[pl.ds(sid * RPT, RPT)])

    @pl.when(cid == 1)
    def _():
        pltpu.sync_copy(xb_h.at[pl.ds(sid * RPT, RPT)],
                        xs_sh.at[pl.ds(sid * RPT, RPT)])

    plsc.subcore_barrier()

    for ph in range(NPH):
        pltpu.sync_copy(src_h.at[pl.ds(base + ph * PHR, PHR)], idxs_v)
        pltpu.sync_copy(dst_h.at[pl.ds(base + ph * PHR, PHR)], idxd_v)
        for b in range(3):
            pltpu.async_copy(xs_sh.at[idxs_v.at[b]], rows[b], gs[b])

        def slot(q, b):
            pltpu.make_async_copy(xs_sh.at[idxs_v.at[q]], rows[b],
                                  gs[b]).wait()
            b2 = (b + 3) % 4

            @pl.when(q + 3 < PHR)
            def _():
                pltpu.async_copy(xs_sh.at[idxs_v.at[q + 3]], rows[b2],
                                 gs[b2])

            pltpu.sync_copy(rows[b], acc_sh.at[idxd_v.at[q]], add=True)

        def body(i, carry):
            for k in range(4):
                slot(i * 4 + k, k)
            return carry

        lax.fori_loop(0, PHR // 4, body, 0)

    plsc.subcore_barrier()

    @pl.when(cid == 0)
    def _():
        pltpu.sync_copy(acc_sh.at[pl.ds(sid * RPT, RPT)],
                        pa_h.at[pl.ds(sid * RPT, RPT)])

    @pl.when(cid == 1)
    def _():
        pltpu.sync_copy(acc_sh.at[pl.ds(sid * RPT, RPT)],
                        pb_h.at[pl.ds(sid * RPT, RPT)])


def _prep_body(x_ref, dgo0_ref, dgo1_ref, dgi0_ref, dgi1_ref,
               xa_ref, xb_ref, so_ref, si_ref):
    dgo = dgo0_ref[...] + dgo1_ref[...]
    dgi = dgi0_ref[...] + dgi1_ref[...]
    so = lax.rsqrt(jnp.maximum(dgo, 1.0))
    si = lax.rsqrt(jnp.maximum(dgi, 1.0))
    so_ref[...] = so
    si_ref[...] = si
    xs = x_ref[...] * so
    xa_ref[...] = xs[:, 0:HD]
    xb_ref[...] = xs[:, HD:D]


def _layer_body(pa_ref, pb_ref, si_ref, so_ref, w_ref, b_ref,
                xa_ref, xb_ref):
    si = si_ref[...]
    z = (jnp.dot(pa_ref[...] * si, w_ref[pl.ds(0, HD), :],
                 preferred_element_type=jnp.float32)
         + jnp.dot(pb_ref[...] * si, w_ref[pl.ds(HD, HD), :],
                   preferred_element_type=jnp.float32))
    res = _selu(z + b_ref[...]) * so_ref[...]
    xa_ref[...] = res[:, 0:HD]
    xb_ref[...] = res[:, HD:D]


def _final_body(pa_ref, pb_ref, si_ref, w3_ref,
                b3_ref, fg_ref, wl1_ref, bl1_ref, wl2_ref, bl2_ref,
                wl3_ref, bl3_ref, out_ref):
    siN = si_ref[pl.ds(0, N), :]
    pa = pa_ref[pl.ds(0, N), :]
    pb = pb_ref[pl.ds(0, N), :]
    h = (jnp.dot(pa * siN, w3_ref[pl.ds(0, HD), :],
                 preferred_element_type=jnp.float32)
         + jnp.dot(pb * siN, w3_ref[pl.ds(HD, HD), :],
                   preferred_element_type=jnp.float32))
    h = h + b3_ref[...]
    emb = jnp.mean(h, axis=0, keepdims=True)
    t = (jnp.dot(emb, wl1_ref[pl.ds(0, H), :],
                 preferred_element_type=jnp.float32)
         + jnp.dot(fg_ref[...], wl1_ref[pl.ds(H, EXTRA), :],
                   preferred_element_type=jnp.float32)
         + bl1_ref[...])
    t = _selu(t)
    t = _selu(jnp.dot(t, wl2_ref[...], preferred_element_type=jnp.float32)
              + bl2_ref[...])
    out_ref[...] = (jnp.dot(t, wl3_ref[...],
                            preferred_element_type=jnp.float32)
                    + bl3_ref[...])


def kernel(edge_index, feats_node, feats_graph, W1, b1, W2, b2, W3, b3,
           Wl1, bl1, Wl2, bl2, Wl3, bl3):
    f32 = jnp.float32
    pad_idx = N + jnp.arange(EPAD - E, dtype=jnp.int32) % (NPAD - N)
    src = jnp.concatenate([edge_index[0], pad_idx]).reshape(NROW, CH)
    dst = jnp.concatenate([edge_index[1], pad_idx]).reshape(NROW, CH)
    x_pad = jnp.concatenate(
        [feats_node, jnp.zeros((NPAD - N, D), f32)], axis=0)
    zeros2d = jnp.zeros((NPAD, HD), f32)

    dgo0, dgo1, dgi0, dgi1 = _deg_kernel(src, dst)

    xa, xb, so, si = pl.pallas_call(
        _prep_body,
        out_shape=[jax.ShapeDtypeStruct((NPAD, HD), f32),
                   jax.ShapeDtypeStruct((NPAD, HD), f32),
                   jax.ShapeDtypeStruct((NPAD, 1), f32),
                   jax.ShapeDtypeStruct((NPAD, 1), f32)],
    )(x_pad, dgo0.reshape(NPAD, 1), dgo1.reshape(NPAD, 1),
      dgi0.reshape(NPAD, 1), dgi1.reshape(NPAD, 1))

    layer = pl.pallas_call(
        _layer_body,
        out_shape=[jax.ShapeDtypeStruct((NPAD, HD), f32),
                   jax.ShapeDtypeStruct((NPAD, HD), f32)],
    )

    for W, b in ((W1, b1), (W2, b2)):
        pa, pb = _agg_kernel(src, dst, xa, xb, zeros2d)
        xa, xb = layer(pa, pb, si, so, W, b.reshape(1, H))

    pa, pb = _agg_kernel(src, dst, xa, xb, zeros2d)
    out = pl.pallas_call(
        _final_body,
        out_shape=jax.ShapeDtypeStruct((1, 1), f32),
    )(pa, pb, si, W3, b3.reshape(1, H), feats_graph, Wl1,
      bl1.reshape(1, 2 * H), Wl2, bl2.reshape(1, H), Wl3, bl3.reshape(1, 1))
    return out.reshape(-1)


# speedup vs baseline: 1.4362x; 1.4362x over previous
import functools

import jax
import jax.numpy as jnp
from jax import lax
from jax.experimental import pallas as pl
from jax.experimental.pallas import tpu as pltpu
from jax.experimental.pallas import tpu_sc as plsc

N = 10000
E = 320000
D = 128
H = 128
EXTRA = 16

NC = 2
NS = 16
CH = 128
EPAD = NC * NS * 80 * CH
NROW = EPAD // CH
NPAD = 10240
RPT = NPAD // NS
DEG_ROWS = NROW // (NC * NS)
AGG_ROWS = NROW // (NC * NS)

_SELU_ALPHA = 1.6732632423543772
_SELU_SCALE = 1.0507009873554805

_MESH = dict(core_axis_name="c", subcore_axis_name="s", num_cores=NC,
             num_subcores=NS)


def _selu(x):
    return _SELU_SCALE * jnp.where(x > 0, x, _SELU_ALPHA * (jnp.exp(x) - 1.0))


@functools.partial(
    pl.kernel,
    out_type=[jax.ShapeDtypeStruct((NPAD,), jnp.float32),
              jax.ShapeDtypeStruct((NPAD,), jnp.float32),
              jax.ShapeDtypeStruct((NPAD,), jnp.float32),
              jax.ShapeDtypeStruct((NPAD,), jnp.float32)],
    mesh=plsc.VectorSubcoreMesh(**_MESH),
    scratch_types=[
        pltpu.VMEM((DEG_ROWS, CH), jnp.int32),
        pltpu.VMEM((DEG_ROWS, CH), jnp.int32),
        pltpu.VMEM((CH,), jnp.float32),
        pltpu.VMEM((RPT,), jnp.float32),
        pltpu.VMEM_SHARED((NPAD,), jnp.float32),
        pltpu.VMEM_SHARED((NPAD,), jnp.float32),
        pltpu.SemaphoreType.DMA,
        pltpu.SemaphoreType.DMA,
    ],
)
def _deg_kernel(src_h, dst_h, dgo0_h, dgo1_h, dgi0_h, dgi1_h,
                idxs_v, idxd_v, ones_v, zbuf_v, dego_sh, degi_sh,
                sso, ssi):
    cid = lax.axis_index("c")
    sid = lax.axis_index("s")
    base = (cid * NS + sid) * DEG_ROWS

    for i in range(CH // 16):
        ones_v[pl.ds(i * 16, 16)] = jnp.ones((16,), jnp.float32)
    for i in range(RPT // 16):
        zbuf_v[pl.ds(i * 16, 16)] = jnp.zeros((16,), jnp.float32)
    pltpu.sync_copy(zbuf_v, dego_sh.at[pl.ds(sid * RPT, RPT)])
    pltpu.sync_copy(zbuf_v, degi_sh.at[pl.ds(sid * RPT, RPT)])
    pltpu.sync_copy(src_h.at[pl.ds(base, DEG_ROWS)], idxs_v)
    pltpu.sync_copy(dst_h.at[pl.ds(base, DEG_ROWS)], idxd_v)
    plsc.subcore_barrier()

    def body(j, carry):
        pltpu.async_copy(ones_v, dego_sh.at[idxs_v.at[j]], sso, add=True)
        pltpu.async_copy(ones_v, degi_sh.at[idxd_v.at[j]], ssi, add=True)
        pltpu.make_async_copy(ones_v, dego_sh.at[idxs_v.at[0]], sso).wait()
        pltpu.make_async_copy(ones_v, degi_sh.at[idxd_v.at[0]], ssi).wait()
        return carry

    lax.fori_loop(0, DEG_ROWS, body, 0)
    plsc.subcore_barrier()

    @pl.when(cid == 0)
    def _():
        pltpu.sync_copy(dego_sh.at[pl.ds(sid * RPT, RPT)],
                        dgo0_h.at[pl.ds(sid * RPT, RPT)])
        pltpu.sync_copy(degi_sh.at[pl.ds(sid * RPT, RPT)],
                        dgi0_h.at[pl.ds(sid * RPT, RPT)])

    @pl.when(cid == 1)
    def _():
        pltpu.sync_copy(dego_sh.at[pl.ds(sid * RPT, RPT)],
                        dgo1_h.at[pl.ds(sid * RPT, RPT)])
        pltpu.sync_copy(degi_sh.at[pl.ds(sid * RPT, RPT)],
                        dgi1_h.at[pl.ds(sid * RPT, RPT)])


@functools.partial(
    pl.kernel,
    out_type=[jax.ShapeDtypeStruct((NPAD, D), jnp.float32),
              jax.ShapeDtypeStruct((NPAD, D), jnp.float32)],
    mesh=plsc.VectorSubcoreMesh(**_MESH),
    scratch_types=[
        pltpu.VMEM((AGG_ROWS // 2, CH), jnp.int32),
        pltpu.VMEM((AGG_ROWS // 2, CH), jnp.int32),
        pltpu.VMEM((CH, D), jnp.float32),
        pltpu.VMEM((CH, D), jnp.float32),
        pltpu.VMEM_SHARED((NPAD, D), jnp.float32),
        pltpu.SemaphoreType.DMA,
        pltpu.SemaphoreType.DMA,
    ],
)
def _agg_kernel(src_h, dst_h, xs_h, zeros_h, p0_h, p1_h, idxs_v, idxd_v,
                rows0_v, rows1_v, agg_sh, gsem0, gsem1):
    cid = lax.axis_index("c")
    sid = lax.axis_index("s")
    base = (cid * NS + sid) * AGG_ROWS
    PH = AGG_ROWS // 2

    pltpu.sync_copy(zeros_h.at[pl.ds(sid * RPT, RPT)],
                    agg_sh.at[pl.ds(sid * RPT, RPT)])

    for phase in range(2):
        pltpu.sync_copy(src_h.at[pl.ds(base + phase * PH, PH)], idxs_v)
        pltpu.sync_copy(dst_h.at[pl.ds(base + phase * PH, PH)], idxd_v)
        if phase == 0:
            plsc.subcore_barrier()
        pltpu.async_copy(xs_h.at[idxs_v.at[0]], rows0_v, gsem0)

        def body(i, carry):
            j = i * 2
            pltpu.async_copy(xs_h.at[idxs_v.at[j + 1]], rows1_v, gsem1)
            pltpu.make_async_copy(xs_h.at[idxs_v.at[j]], rows0_v,
                                  gsem0).wait()
            pltpu.sync_copy(rows0_v, agg_sh.at[idxd_v.at[j]], add=True)

            @pl.when(j + 2 < PH)
            def _():
                pltpu.async_copy(xs_h.at[idxs_v.at[j + 2]], rows0_v, gsem0)

            pltpu.make_async_copy(xs_h.at[idxs_v.at[j + 1]], rows1_v,
                                  gsem1).wait()
            pltpu.sync_copy(rows1_v, agg_sh.at[idxd_v.at[j + 1]], add=True)
            return carry

        lax.fori_loop(0, PH // 2, body, 0)

    plsc.subcore_barrier()

    @pl.when(cid == 0)
    def _():
        pltpu.sync_copy(agg_sh.at[pl.ds(sid * RPT, RPT)],
                        p0_h.at[pl.ds(sid * RPT, RPT)])

    @pl.when(cid == 1)
    def _():
        pltpu.sync_copy(agg_sh.at[pl.ds(sid * RPT, RPT)],
                        p1_h.at[pl.ds(sid * RPT, RPT)])


def _prep_body(x_ref, dgo0_ref, dgo1_ref, dgi0_ref, dgi1_ref,
               xs_ref, so_ref, si_ref):
    dgo = dgo0_ref[...] + dgo1_ref[...]
    dgi = dgi0_ref[...] + dgi1_ref[...]
    so = lax.rsqrt(jnp.maximum(dgo, 1.0))
    si = lax.rsqrt(jnp.maximum(dgi, 1.0))
    so_ref[...] = so
    si_ref[...] = si
    xs_ref[...] = x_ref[...] * so


def _layer_body(p0_ref, p1_ref, si_ref, so_ref, w_ref, b_ref, out_ref):
    agg = (p0_ref[...] + p1_ref[...]) * si_ref[...]
    z = jnp.dot(agg, w_ref[...], preferred_element_type=jnp.float32)
    out_ref[...] = _selu(z + b_ref[...]) * so_ref[...]


def _final_body(p0_ref, p1_ref, si_ref, w3_ref, b3_ref, fg_ref, wl1_ref,
                bl1_ref, wl2_ref, bl2_ref, wl3_ref, bl3_ref, out_ref):
    agg = (p0_ref[pl.ds(0, N), :] + p1_ref[pl.ds(0, N), :]) \
        * si_ref[pl.ds(0, N), :]
    h = jnp.dot(agg, w3_ref[...], preferred_element_type=jnp.float32)
    h = h + b3_ref[...]
    emb = jnp.mean(h, axis=0, keepdims=True)
    t = (jnp.dot(emb, wl1_ref[pl.ds(0, H), :],
                 preferred_element_type=jnp.float32)
         + jnp.dot(fg_ref[...], wl1_ref[pl.ds(H, EXTRA), :],
                   preferred_element_type=jnp.float32)
         + bl1_ref[...])
    t = _selu(t)
    t = _selu(jnp.dot(t, wl2_ref[...], preferred_element_type=jnp.float32)
              + bl2_ref[...])
    out_ref[...] = (jnp.dot(t, wl3_ref[...],
                            preferred_element_type=jnp.float32)
                    + bl3_ref[...])


def kernel(edge_index, feats_node, feats_graph, W1, b1, W2, b2, W3, b3,
           Wl1, bl1, Wl2, bl2, Wl3, bl3):
    f32 = jnp.float32
    pad_idx = N + jnp.arange(EPAD - E, dtype=jnp.int32) % (NPAD - N)
    src = jnp.concatenate([edge_index[0], pad_idx]).reshape(NROW, CH)
    dst = jnp.concatenate([edge_index[1], pad_idx]).reshape(NROW, CH)
    x_pad = jnp.concatenate(
        [feats_node, jnp.zeros((NPAD - N, D), f32)], axis=0)
    zeros2d = jnp.zeros((NPAD, D), f32)

    dgo0, dgo1, dgi0, dgi1 = _deg_kernel(src, dst)

    xs0, so, si = pl.pallas_call(
        _prep_body,
        out_shape=[jax.ShapeDtypeStruct((NPAD, D), f32),
                   jax.ShapeDtypeStruct((NPAD, 1), f32),
                   jax.ShapeDtypeStruct((NPAD, 1), f32)],
    )(x_pad, dgo0.reshape(NPAD, 1), dgo1.reshape(NPAD, 1),
      dgi0.reshape(NPAD, 1), dgi1.reshape(NPAD, 1))

    layer = pl.pallas_call(
        _layer_body,
        out_shape=jax.ShapeDtypeStruct((NPAD, D), f32),
    )

    x = xs0
    for W, b in ((W1, b1), (W2, b2)):
        pa, pb = _agg_kernel(src, dst, x, zeros2d)
        x = layer(pa, pb, si, so, W, b.reshape(1, H))

    pa, pb = _agg_kernel(src, dst, x, zeros2d)
    out = pl.pallas_call(
        _final_body,
        out_shape=jax.ShapeDtypeStruct((1, 1), f32),
    )(pa, pb, si, W3, b3.reshape(1, H), feats_graph, Wl1,
      bl1.reshape(1, 2 * H), Wl2, bl2.reshape(1, H), Wl3, bl3.reshape(1, 1))
    return out.reshape(-1)


# speedup vs baseline: 1.4411x; 1.0034x over previous
import functools

import jax
import jax.numpy as jnp
from jax import lax
from jax.experimental import pallas as pl
from jax.experimental.pallas import tpu as pltpu
from jax.experimental.pallas import tpu_sc as plsc

N = 10000
E = 320000
D = 128
H = 128
EXTRA = 16

NC = 2
NS = 16
CH = 128
EPAD = NC * NS * 80 * CH
NROW = EPAD // CH
NPAD = 10240
RPT = NPAD // NS
DEG_ROWS = NROW // (NC * NS)
AGG_ROWS = NROW // (NC * NS)

_SELU_ALPHA = 1.6732632423543772
_SELU_SCALE = 1.0507009873554805

_MESH = dict(core_axis_name="c", subcore_axis_name="s", num_cores=NC,
             num_subcores=NS)


def _selu(x):
    return _SELU_SCALE * jnp.where(x > 0, x, _SELU_ALPHA * (jnp.exp(x) - 1.0))


@functools.partial(
    pl.kernel,
    out_type=[jax.ShapeDtypeStruct((2 * NPAD,), jnp.float32),
              jax.ShapeDtypeStruct((2 * NPAD,), jnp.float32)],
    mesh=plsc.VectorSubcoreMesh(**_MESH),
    scratch_types=[
        pltpu.VMEM((DEG_ROWS, CH), jnp.int32),
        pltpu.VMEM((DEG_ROWS, CH), jnp.int32),
        pltpu.VMEM((CH,), jnp.float32),
        pltpu.VMEM((RPT,), jnp.float32),
        pltpu.VMEM_SHARED((NPAD,), jnp.float32),
        pltpu.VMEM_SHARED((NPAD,), jnp.float32),
        pltpu.SemaphoreType.DMA,
        pltpu.SemaphoreType.DMA,
    ],
)
def _deg_kernel(src_h, dst_h, dg0_h, dg1_h,
                idxs_v, idxd_v, ones_v, zbuf_v, dego_sh, degi_sh,
                sso, ssi):
    cid = lax.axis_index("c")
    sid = lax.axis_index("s")
    base = (cid * NS + sid) * DEG_ROWS

    for i in range(CH // 16):
        ones_v[pl.ds(i * 16, 16)] = jnp.ones((16,), jnp.float32)
    for i in range(RPT // 16):
        zbuf_v[pl.ds(i * 16, 16)] = jnp.zeros((16,), jnp.float32)
    pltpu.sync_copy(zbuf_v, dego_sh.at[pl.ds(sid * RPT, RPT)])
    pltpu.sync_copy(zbuf_v, degi_sh.at[pl.ds(sid * RPT, RPT)])
    pltpu.sync_copy(src_h.at[pl.ds(base, DEG_ROWS)], idxs_v)
    pltpu.sync_copy(dst_h.at[pl.ds(base, DEG_ROWS)], idxd_v)
    plsc.subcore_barrier()

    def body(j, carry):
        pltpu.async_copy(ones_v, dego_sh.at[idxs_v.at[j]], sso, add=True)
        pltpu.async_copy(ones_v, degi_sh.at[idxd_v.at[j]], ssi, add=True)
        pltpu.make_async_copy(ones_v, dego_sh.at[idxs_v.at[0]], sso).wait()
        pltpu.make_async_copy(ones_v, degi_sh.at[idxd_v.at[0]], ssi).wait()
        return carry

    lax.fori_loop(0, DEG_ROWS, body, 0)
    plsc.subcore_barrier()

    @pl.when(cid == 0)
    def _():
        pltpu.sync_copy(dego_sh.at[pl.ds(sid * RPT, RPT)],
                        dg0_h.at[pl.ds(sid * RPT, RPT)])
        pltpu.sync_copy(degi_sh.at[pl.ds(sid * RPT, RPT)],
                        dg0_h.at[pl.ds(NPAD + sid * RPT, RPT)])

    @pl.when(cid == 1)
    def _():
        pltpu.sync_copy(dego_sh.at[pl.ds(sid * RPT, RPT)],
                        dg1_h.at[pl.ds(sid * RPT, RPT)])
        pltpu.sync_copy(degi_sh.at[pl.ds(sid * RPT, RPT)],
                        dg1_h.at[pl.ds(NPAD + sid * RPT, RPT)])


@functools.partial(
    pl.kernel,
    out_type=[jax.ShapeDtypeStruct((NPAD, D), jnp.float32),
              jax.ShapeDtypeStruct((NPAD, D), jnp.float32)],
    mesh=plsc.VectorSubcoreMesh(**_MESH),
    scratch_types=[
        pltpu.VMEM((AGG_ROWS // 2, CH), jnp.int32),
        pltpu.VMEM((AGG_ROWS // 2, CH), jnp.int32),
        pltpu.VMEM((CH, D), jnp.float32),
        pltpu.VMEM((CH, D), jnp.float32),
        pltpu.VMEM_SHARED((NPAD, D), jnp.float32),
        pltpu.SemaphoreType.DMA,
        pltpu.SemaphoreType.DMA,
    ],
)
def _agg_kernel(src_h, dst_h, xs_h, zeros_h, p0_h, p1_h, idxs_v, idxd_v,
                rows0_v, rows1_v, agg_sh, gsem0, gsem1):
    cid = lax.axis_index("c")
    sid = lax.axis_index("s")
    base = (cid * NS + sid) * AGG_ROWS
    PH = AGG_ROWS // 2

    pltpu.sync_copy(zeros_h.at[pl.ds(sid * RPT, RPT)],
                    agg_sh.at[pl.ds(sid * RPT, RPT)])

    for phase in range(2):
        pltpu.sync_copy(src_h.at[pl.ds(base + phase * PH, PH)], idxs_v)
        pltpu.sync_copy(dst_h.at[pl.ds(base + phase * PH, PH)], idxd_v)
        if phase == 0:
            plsc.subcore_barrier()
        pltpu.async_copy(xs_h.at[idxs_v.at[0]], rows0_v, gsem0)

        def body(i, carry):
            j = i * 2
            pltpu.async_copy(xs_h.at[idxs_v.at[j + 1]], rows1_v, gsem1)
            pltpu.make_async_copy(xs_h.at[idxs_v.at[j]], rows0_v,
                                  gsem0).wait()
            pltpu.sync_copy(rows0_v, agg_sh.at[idxd_v.at[j]], add=True)

            @pl.when(j + 2 < PH)
            def _():
                pltpu.async_copy(xs_h.at[idxs_v.at[j + 2]], rows0_v, gsem0)

            pltpu.make_async_copy(xs_h.at[idxs_v.at[j + 1]], rows1_v,
                                  gsem1).wait()
            pltpu.sync_copy(rows1_v, agg_sh.at[idxd_v.at[j + 1]], add=True)
            return carry

        lax.fori_loop(0, PH // 2, body, 0)

    plsc.subcore_barrier()

    @pl.when(cid == 0)
    def _():
        pltpu.sync_copy(agg_sh.at[pl.ds(sid * RPT, RPT)],
                        p0_h.at[pl.ds(sid * RPT, RPT)])

    @pl.when(cid == 1)
    def _():
        pltpu.sync_copy(agg_sh.at[pl.ds(sid * RPT, RPT)],
                        p1_h.at[pl.ds(sid * RPT, RPT)])


GB = 8
RB = NPAD // GB


def _prep_body(x_ref, dgo0_ref, dgo1_ref, dgi0_ref, dgi1_ref,
               xs_ref, so_ref, si_ref):
    dgo = dgo0_ref[...] + dgo1_ref[...]
    dgi = dgi0_ref[...] + dgi1_ref[...]
    so = lax.rsqrt(jnp.maximum(dgo, 1.0))
    si = lax.rsqrt(jnp.maximum(dgi, 1.0))
    so_ref[...] = so
    si_ref[...] = si
    xs_ref[...] = x_ref[...] * so


def _layer_body(p0_ref, p1_ref, si_ref, so_ref, w_ref, b_ref, out_ref):
    agg = (p0_ref[...] + p1_ref[...]) * si_ref[...]
    z = jnp.dot(agg, w_ref[...], preferred_element_type=jnp.float32)
    out_ref[...] = _selu(z + b_ref[...]) * so_ref[...]


def _final_body(p0_ref, p1_ref, si_ref, w3_ref, b3_ref, fg_ref, wl1_ref,
                bl1_ref, wl2_ref, bl2_ref, wl3_ref, bl3_ref, out_ref):
    agg = (p0_ref[pl.ds(0, N), :] + p1_ref[pl.ds(0, N), :]) \
        * si_ref[pl.ds(0, N), :]
    h = jnp.dot(agg, w3_ref[...], preferred_element_type=jnp.float32)
    h = h + b3_ref[...]
    emb = jnp.mean(h, axis=0, keepdims=True)
    t = (jnp.dot(emb, wl1_ref[pl.ds(0, H), :],
                 preferred_element_type=jnp.float32)
         + jnp.dot(fg_ref[...], wl1_ref[pl.ds(H, EXTRA), :],
                   preferred_element_type=jnp.float32)
         + bl1_ref[...])
    t = _selu(t)
    t = _selu(jnp.dot(t, wl2_ref[...], preferred_element_type=jnp.float32)
              + bl2_ref[...])
    out_ref[...] = (jnp.dot(t, wl3_ref[...],
                            preferred_element_type=jnp.float32)
                    + bl3_ref[...])


def kernel(edge_index, feats_node, feats_graph, W1, b1, W2, b2, W3, b3,
           Wl1, bl1, Wl2, bl2, Wl3, bl3):
    f32 = jnp.float32
    pad_idx = N + jnp.arange(EPAD - E, dtype=jnp.int32) % (NPAD - N)
    src = jnp.concatenate([edge_index[0], pad_idx]).reshape(NROW, CH)
    dst = jnp.concatenate([edge_index[1], pad_idx]).reshape(NROW, CH)
    x_pad = jnp.concatenate(
        [feats_node, jnp.zeros((NPAD - N, D), f32)], axis=0)
    zeros2d = jnp.zeros((NPAD, D), f32)

    dg0, dg1 = _deg_kernel(src, dst)
    dg0 = dg0.reshape(2 * NPAD, 1)
    dg1 = dg1.reshape(2 * NPAD, 1)

    colv = lambda off: pl.BlockSpec((RB, 1), lambda i, off=off: (i + off, 0))
    rowsD = pl.BlockSpec((RB, D), lambda i: (i, 0))
    whole = lambda s: pl.BlockSpec(s, lambda i: (0, 0))

    xs0, so, si = pl.pallas_call(
        _prep_body,
        grid=(GB,),
        in_specs=[rowsD, colv(0), colv(0), colv(GB), colv(GB)],
        out_specs=[rowsD, colv(0), colv(0)],
        out_shape=[jax.ShapeDtypeStruct((NPAD, D), f32),
                   jax.ShapeDtypeStruct((NPAD, 1), f32),
                   jax.ShapeDtypeStruct((NPAD, 1), f32)],
    )(x_pad, dg0, dg1, dg0, dg1)

    layer = pl.pallas_call(
        _layer_body,
        grid=(GB,),
        in_specs=[rowsD, rowsD, colv(0), colv(0),
                  whole((D, H)), whole((1, H))],
        out_specs=rowsD,
        out_shape=jax.ShapeDtypeStruct((NPAD, D), f32),
    )

    x = xs0
    for W, b in ((W1, b1), (W2, b2)):
        pa, pb = _agg_kernel(src, dst, x, zeros2d)
        x = layer(pa, pb, si, so, W, b.reshape(1, H))

    pa, pb = _agg_kernel(src, dst, x, zeros2d)
    out = pl.pallas_call(
        _final_body,
        out_shape=jax.ShapeDtypeStruct((1, 1), f32),
    )(pa, pb, si, W3, b3.reshape(1, H), feats_graph, Wl1,
      bl1.reshape(1, 2 * H), Wl2, bl2.reshape(1, H), Wl3, bl3.reshape(1, 1))
    return out.reshape(-1)


# speedup vs baseline: 1.4589x; 1.0123x over previous
import functools

import jax
import jax.numpy as jnp
from jax import lax
from jax.experimental import pallas as pl
from jax.experimental.pallas import tpu as pltpu
from jax.experimental.pallas import tpu_sc as plsc

N = 10000
E = 320000
D = 128
H = 128
EXTRA = 16

NC = 2
NS = 16
CH = 128
EPAD = NC * NS * 80 * CH
NROW = EPAD // CH
NPAD = 10240
RPT = NPAD // NS
DEG_ROWS = NROW // (NC * NS)
AGG_ROWS = NROW // (NC * NS)

_SELU_ALPHA = 1.6732632423543772
_SELU_SCALE = 1.0507009873554805

_MESH = dict(core_axis_name="c", subcore_axis_name="s", num_cores=NC,
             num_subcores=NS)


def _selu(x):
    return _SELU_SCALE * jnp.where(x > 0, x, _SELU_ALPHA * (jnp.exp(x) - 1.0))


@functools.partial(
    pl.kernel,
    out_type=[jax.ShapeDtypeStruct((2 * NPAD,), jnp.float32),
              jax.ShapeDtypeStruct((2 * NPAD,), jnp.float32)],
    mesh=plsc.VectorSubcoreMesh(**_MESH),
    scratch_types=[
        pltpu.VMEM((DEG_ROWS, CH), jnp.int32),
        pltpu.VMEM((DEG_ROWS, CH), jnp.int32),
        pltpu.VMEM((CH,), jnp.float32),
        pltpu.VMEM((RPT,), jnp.float32),
        pltpu.VMEM_SHARED((NPAD,), jnp.float32),
        pltpu.VMEM_SHARED((NPAD,), jnp.float32),
        pltpu.SemaphoreType.DMA,
        pltpu.SemaphoreType.DMA,
    ],
)
def _deg_kernel(src_h, dst_h, dg0_h, dg1_h,
                idxs_v, idxd_v, ones_v, zbuf_v, dego_sh, degi_sh,
                sso, ssi):
    cid = lax.axis_index("c")
    sid = lax.axis_index("s")
    base = (cid * NS + sid) * DEG_ROWS

    for i in range(CH // 16):
        ones_v[pl.ds(i * 16, 16)] = jnp.ones((16,), jnp.float32)
    for i in range(RPT // 16):
        zbuf_v[pl.ds(i * 16, 16)] = jnp.zeros((16,), jnp.float32)
    pltpu.sync_copy(zbuf_v, dego_sh.at[pl.ds(sid * RPT, RPT)])
    pltpu.sync_copy(zbuf_v, degi_sh.at[pl.ds(sid * RPT, RPT)])
    pltpu.sync_copy(src_h.at[pl.ds(base, DEG_ROWS)], idxs_v)
    pltpu.sync_copy(dst_h.at[pl.ds(base, DEG_ROWS)], idxd_v)
    plsc.subcore_barrier()

    def body(j, carry):
        pltpu.async_copy(ones_v, dego_sh.at[idxs_v.at[j]], sso, add=True)
        pltpu.async_copy(ones_v, degi_sh.at[idxd_v.at[j]], ssi, add=True)
        pltpu.make_async_copy(ones_v, dego_sh.at[idxs_v.at[0]], sso).wait()
        pltpu.make_async_copy(ones_v, degi_sh.at[idxd_v.at[0]], ssi).wait()
        return carry

    lax.fori_loop(0, DEG_ROWS, body, 0)
    plsc.subcore_barrier()

    @pl.when(cid == 0)
    def _():
        pltpu.sync_copy(dego_sh.at[pl.ds(sid * RPT, RPT)],
                        dg0_h.at[pl.ds(sid * RPT, RPT)])
        pltpu.sync_copy(degi_sh.at[pl.ds(sid * RPT, RPT)],
                        dg0_h.at[pl.ds(NPAD + sid * RPT, RPT)])

    @pl.when(cid == 1)
    def _():
        pltpu.sync_copy(dego_sh.at[pl.ds(sid * RPT, RPT)],
                        dg1_h.at[pl.ds(sid * RPT, RPT)])
        pltpu.sync_copy(degi_sh.at[pl.ds(sid * RPT, RPT)],
                        dg1_h.at[pl.ds(NPAD + sid * RPT, RPT)])


@functools.partial(
    pl.kernel,
    out_type=[jax.ShapeDtypeStruct((NPAD, D), jnp.float32),
              jax.ShapeDtypeStruct((NPAD, D), jnp.float32)],
    mesh=plsc.VectorSubcoreMesh(**_MESH),
    scratch_types=[
        pltpu.VMEM((AGG_ROWS // 2, CH), jnp.int32),
        pltpu.VMEM((AGG_ROWS // 2, CH), jnp.int32),
        pltpu.VMEM((CH, D), jnp.float32),
        pltpu.VMEM((CH, D), jnp.float32),
        pltpu.VMEM_SHARED((NPAD, D), jnp.float32),
        pltpu.SemaphoreType.DMA,
        pltpu.SemaphoreType.DMA,
    ],
)
def _agg_kernel(src_h, dst_h, xs_h, zeros_h, p0_h, p1_h, idxs_v, idxd_v,
                rows0_v, rows1_v, agg_sh, gsem0, gsem1):
    cid = lax.axis_index("c")
    sid = lax.axis_index("s")
    base = (cid * NS + sid) * AGG_ROWS
    PH = AGG_ROWS // 2

    for phase in range(2):
        pltpu.sync_copy(src_h.at[pl.ds(base + phase * PH, PH)], idxs_v)
        pltpu.sync_copy(dst_h.at[pl.ds(base + phase * PH, PH)], idxd_v)
        pltpu.async_copy(xs_h.at[idxs_v.at[0]], rows0_v, gsem0)
        if phase == 0:
            pltpu.sync_copy(zeros_h.at[pl.ds(sid * RPT, RPT)],
                            agg_sh.at[pl.ds(sid * RPT, RPT)])
            plsc.subcore_barrier()

        def body(i, carry):
            j = i * 2
            pltpu.async_copy(xs_h.at[idxs_v.at[j + 1]], rows1_v, gsem1)
            pltpu.make_async_copy(xs_h.at[idxs_v.at[j]], rows0_v,
                                  gsem0).wait()
            pltpu.sync_copy(rows0_v, agg_sh.at[idxd_v.at[j]], add=True)

            @pl.when(j + 2 < PH)
            def _():
                pltpu.async_copy(xs_h.at[idxs_v.at[j + 2]], rows0_v, gsem0)

            pltpu.make_async_copy(xs_h.at[idxs_v.at[j + 1]], rows1_v,
                                  gsem1).wait()
            pltpu.sync_copy(rows1_v, agg_sh.at[idxd_v.at[j + 1]], add=True)
            return carry

        lax.fori_loop(0, PH // 2, body, 0)

    plsc.subcore_barrier()

    @pl.when(cid == 0)
    def _():
        pltpu.sync_copy(agg_sh.at[pl.ds(sid * RPT, RPT)],
                        p0_h.at[pl.ds(sid * RPT, RPT)])

    @pl.when(cid == 1)
    def _():
        pltpu.sync_copy(agg_sh.at[pl.ds(sid * RPT, RPT)],
                        p1_h.at[pl.ds(sid * RPT, RPT)])


GB = 8
RB = NPAD // GB


def _prep_body(x_ref, dgo0_ref, dgo1_ref, dgi0_ref, dgi1_ref,
               xs_ref, so_ref, si_ref):
    dgo = dgo0_ref[...] + dgo1_ref[...]
    dgi = dgi0_ref[...] + dgi1_ref[...]
    so = lax.rsqrt(jnp.maximum(dgo, 1.0))
    si = lax.rsqrt(jnp.maximum(dgi, 1.0))
    so_ref[...] = so
    si_ref[...] = si
    xs_ref[...] = x_ref[...] * so


def _layer_body(p0_ref, p1_ref, si_ref, so_ref, w_ref, b_ref, out_ref):
    agg = (p0_ref[...] + p1_ref[...]) * si_ref[...]
    z = jnp.dot(agg, w_ref[...], preferred_element_type=jnp.float32)
    out_ref[...] = _selu(z + b_ref[...]) * so_ref[...]


def _final_body(p0_ref, p1_ref, si_ref, w3_ref, b3_ref, fg_ref, wl1_ref,
                bl1_ref, wl2_ref, bl2_ref, wl3_ref, bl3_ref, out_ref):
    agg = (p0_ref[pl.ds(0, N), :] + p1_ref[pl.ds(0, N), :]) \
        * si_ref[pl.ds(0, N), :]
    h = jnp.dot(agg, w3_ref[...], preferred_element_type=jnp.float32)
    h = h + b3_ref[...]
    emb = jnp.mean(h, axis=0, keepdims=True)
    t = (jnp.dot(emb, wl1_ref[pl.ds(0, H), :],
                 preferred_element_type=jnp.float32)
         + jnp.dot(fg_ref[...], wl1_ref[pl.ds(H, EXTRA), :],
                   preferred_element_type=jnp.float32)
         + bl1_ref[...])
    t = _selu(t)
    t = _selu(jnp.dot(t, wl2_ref[...], preferred_element_type=jnp.float32)
              + bl2_ref[...])
    out_ref[...] = (jnp.dot(t, wl3_ref[...],
                            preferred_element_type=jnp.float32)
                    + bl3_ref[...])


def kernel(edge_index, feats_node, feats_graph, W1, b1, W2, b2, W3, b3,
           Wl1, bl1, Wl2, bl2, Wl3, bl3):
    f32 = jnp.float32
    pad_idx = N + jnp.arange(EPAD - E, dtype=jnp.int32) % (NPAD - N)
    src = jnp.concatenate([edge_index[0], pad_idx]).reshape(NROW, CH)
    dst = jnp.concatenate([edge_index[1], pad_idx]).reshape(NROW, CH)
    x_pad = jnp.concatenate(
        [feats_node, jnp.zeros((NPAD - N, D), f32)], axis=0)
    zeros2d = jnp.zeros((NPAD, D), f32)

    dg0, dg1 = _deg_kernel(src, dst)
    dg0 = dg0.reshape(2 * NPAD, 1)
    dg1 = dg1.reshape(2 * NPAD, 1)

    colv = lambda off: pl.BlockSpec((RB, 1), lambda i, off=off: (i + off, 0))
    rowsD = pl.BlockSpec((RB, D), lambda i: (i, 0))
    whole = lambda s: pl.BlockSpec(s, lambda i: (0, 0))

    xs0, so, si = pl.pallas_call(
        _prep_body,
        grid=(GB,),
        in_specs=[rowsD, colv(0), colv(0), colv(GB), colv(GB)],
        out_specs=[rowsD, colv(0), colv(0)],
        out_shape=[jax.ShapeDtypeStruct((NPAD, D), f32),
                   jax.ShapeDtypeStruct((NPAD, 1), f32),
                   jax.ShapeDtypeStruct((NPAD, 1), f32)],
    )(x_pad, dg0, dg1, dg0, dg1)

    layer = pl.pallas_call(
        _layer_body,
        grid=(GB,),
        in_specs=[rowsD, rowsD, colv(0), colv(0),
                  whole((D, H)), whole((1, H))],
        out_specs=rowsD,
        out_shape=jax.ShapeDtypeStruct((NPAD, D), f32),
    )

    x = xs0
    for W, b in ((W1, b1), (W2, b2)):
        pa, pb = _agg_kernel(src, dst, x, zeros2d)
        x = layer(pa, pb, si, so, W, b.reshape(1, H))

    pa, pb = _agg_kernel(src, dst, x, zeros2d)
    out = pl.pallas_call(
        _final_body,
        out_shape=jax.ShapeDtypeStruct((1, 1), f32),
    )(pa, pb, si, W3, b3.reshape(1, H), feats_graph, Wl1,
      bl1.reshape(1, 2 * H), Wl2, bl2.reshape(1, H), Wl3, bl3.reshape(1, 1))
    return out.reshape(-1)


# speedup vs baseline: 1.4730x; 1.0096x over previous
import functools

import jax
import jax.numpy as jnp
from jax import lax
from jax.experimental import pallas as pl
from jax.experimental.pallas import tpu as pltpu
from jax.experimental.pallas import tpu_sc as plsc

N = 10000
E = 320000
D = 128
H = 128
EXTRA = 16

NC = 2
NS = 16
CH = 128
EPAD = NC * NS * 80 * CH
NROW = EPAD // CH
NPAD = 10240
RPT = NPAD // NS
DEG_ROWS = NROW // (NC * NS)
AGG_ROWS = NROW // (NC * NS)

_SELU_ALPHA = 1.6732632423543772
_SELU_SCALE = 1.0507009873554805

_MESH = dict(core_axis_name="c", subcore_axis_name="s", num_cores=NC,
             num_subcores=NS)


def _selu(x):
    return _SELU_SCALE * jnp.where(x > 0, x, _SELU_ALPHA * (jnp.exp(x) - 1.0))


@functools.partial(
    pl.kernel,
    out_type=[jax.ShapeDtypeStruct((2 * NPAD,), jnp.float32),
              jax.ShapeDtypeStruct((2 * NPAD,), jnp.float32)],
    mesh=plsc.VectorSubcoreMesh(**_MESH),
    scratch_types=[
        pltpu.VMEM((DEG_ROWS, CH), jnp.int32),
        pltpu.VMEM((DEG_ROWS, CH), jnp.int32),
        pltpu.VMEM((CH,), jnp.float32),
        pltpu.VMEM((RPT,), jnp.float32),
        pltpu.VMEM_SHARED((NPAD,), jnp.float32),
        pltpu.VMEM_SHARED((NPAD,), jnp.float32),
        pltpu.SemaphoreType.DMA,
        pltpu.SemaphoreType.DMA,
    ],
)
def _deg_kernel(src_h, dst_h, dg0_h, dg1_h,
                idxs_v, idxd_v, ones_v, zbuf_v, dego_sh, degi_sh,
                sso, ssi):
    cid = lax.axis_index("c")
    sid = lax.axis_index("s")
    base = (cid * NS + sid) * DEG_ROWS

    for i in range(CH // 16):
        ones_v[pl.ds(i * 16, 16)] = jnp.ones((16,), jnp.float32)
    for i in range(RPT // 16):
        zbuf_v[pl.ds(i * 16, 16)] = jnp.zeros((16,), jnp.float32)
    pltpu.sync_copy(zbuf_v, dego_sh.at[pl.ds(sid * RPT, RPT)])
    pltpu.sync_copy(zbuf_v, degi_sh.at[pl.ds(sid * RPT, RPT)])
    pltpu.sync_copy(src_h.at[pl.ds(base, DEG_ROWS)], idxs_v)
    pltpu.sync_copy(dst_h.at[pl.ds(base, DEG_ROWS)], idxd_v)
    plsc.subcore_barrier()

    def body(j, carry):
        pltpu.async_copy(ones_v, dego_sh.at[idxs_v.at[j]], sso, add=True)
        pltpu.async_copy(ones_v, degi_sh.at[idxd_v.at[j]], ssi, add=True)
        pltpu.make_async_copy(ones_v, dego_sh.at[idxs_v.at[0]], sso).wait()
        pltpu.make_async_copy(ones_v, degi_sh.at[idxd_v.at[0]], ssi).wait()
        return carry

    lax.fori_loop(0, DEG_ROWS, body, 0)
    plsc.subcore_barrier()

    @pl.when(cid == 0)
    def _():
        pltpu.sync_copy(dego_sh.at[pl.ds(sid * RPT, RPT)],
                        dg0_h.at[pl.ds(sid * RPT, RPT)])
        pltpu.sync_copy(degi_sh.at[pl.ds(sid * RPT, RPT)],
                        dg0_h.at[pl.ds(NPAD + sid * RPT, RPT)])

    @pl.when(cid == 1)
    def _():
        pltpu.sync_copy(dego_sh.at[pl.ds(sid * RPT, RPT)],
                        dg1_h.at[pl.ds(sid * RPT, RPT)])
        pltpu.sync_copy(degi_sh.at[pl.ds(sid * RPT, RPT)],
                        dg1_h.at[pl.ds(NPAD + sid * RPT, RPT)])


@functools.partial(
    pl.kernel,
    out_type=[jax.ShapeDtypeStruct((NPAD, D), jnp.float32),
              jax.ShapeDtypeStruct((NPAD, D), jnp.float32)],
    mesh=plsc.VectorSubcoreMesh(**_MESH),
    scratch_types=[
        pltpu.VMEM((AGG_ROWS // 2, CH), jnp.int32),
        pltpu.VMEM((AGG_ROWS // 2, CH), jnp.int32),
        pltpu.VMEM((CH, D), jnp.float32),
        pltpu.VMEM((CH, D), jnp.float32),
        pltpu.VMEM_SHARED((NPAD, D), jnp.float32),
        pltpu.SemaphoreType.DMA,
        pltpu.SemaphoreType.DMA,
    ],
)
def _agg_kernel(src_h, dst_h, xs_h, zeros_h, p0_h, p1_h, idxs_v, idxd_v,
                rows0_v, rows1_v, agg_sh, gsem0, gsem1):
    cid = lax.axis_index("c")
    sid = lax.axis_index("s")
    base = (cid * NS + sid) * AGG_ROWS
    PH = AGG_ROWS // 2

    for phase in range(2):
        pltpu.sync_copy(src_h.at[pl.ds(base + phase * PH, PH)], idxs_v)
        pltpu.sync_copy(dst_h.at[pl.ds(base + phase * PH, PH)], idxd_v)
        pltpu.async_copy(xs_h.at[idxs_v.at[0]], rows0_v, gsem0)
        if phase == 0:
            pltpu.sync_copy(zeros_h.at[pl.ds(sid * RPT, RPT)],
                            agg_sh.at[pl.ds(sid * RPT, RPT)])
            plsc.subcore_barrier()

        def body(i, carry):
            j = i * 2
            pltpu.async_copy(xs_h.at[idxs_v.at[j + 1]], rows1_v, gsem1)
            pltpu.make_async_copy(xs_h.at[idxs_v.at[j]], rows0_v,
                                  gsem0).wait()
            pltpu.sync_copy(rows0_v, agg_sh.at[idxd_v.at[j]], add=True)

            @pl.when(j + 2 < PH)
            def _():
                pltpu.async_copy(xs_h.at[idxs_v.at[j + 2]], rows0_v, gsem0)

            pltpu.make_async_copy(xs_h.at[idxs_v.at[j + 1]], rows1_v,
                                  gsem1).wait()
            pltpu.sync_copy(rows1_v, agg_sh.at[idxd_v.at[j + 1]], add=True)
            return carry

        lax.fori_loop(0, PH // 2, body, 0)

    plsc.subcore_barrier()

    @pl.when(cid == 0)
    def _():
        pltpu.sync_copy(agg_sh.at[pl.ds(sid * RPT, RPT)],
                        p0_h.at[pl.ds(sid * RPT, RPT)])

    @pl.when(cid == 1)
    def _():
        pltpu.sync_copy(agg_sh.at[pl.ds(sid * RPT, RPT)],
                        p1_h.at[pl.ds(sid * RPT, RPT)])


GB = 8
RB = NPAD // GB


def _prep_body(x_ref, dgo0_ref, dgo1_ref, dgi0_ref, dgi1_ref,
               xs_ref, sc_ref):
    dgo = dgo0_ref[...] + dgo1_ref[...]
    dgi = dgi0_ref[...] + dgi1_ref[...]
    so = lax.rsqrt(jnp.maximum(dgo, 1.0))
    si = lax.rsqrt(jnp.maximum(dgi, 1.0))
    sc_ref[:, 0:1] = si
    sc_ref[:, 1:2] = so
    xs_ref[...] = x_ref[...] * so


def _layer_body(p0_ref, p1_ref, sc_ref, w_ref, b_ref, out_ref):
    agg = (p0_ref[...] + p1_ref[...]) * sc_ref[:, 0:1]
    z = jnp.dot(agg, w_ref[...], preferred_element_type=jnp.float32)
    out_ref[...] = _selu(z + b_ref[...]) * sc_ref[:, 1:2]


def _final_body(p0_ref, p1_ref, sc_ref, w3_ref, b3_ref, fg_ref, wl1_ref,
                bl1_ref, wl2_ref, bl2_ref, wl3_ref, bl3_ref, out_ref):
    agg = (p0_ref[pl.ds(0, N), :] + p1_ref[pl.ds(0, N), :]) \
        * sc_ref[pl.ds(0, N), 0:1]
    h = jnp.dot(agg, w3_ref[...], preferred_element_type=jnp.float32)
    h = h + b3_ref[...]
    emb = jnp.mean(h, axis=0, keepdims=True)
    t = (jnp.dot(emb, wl1_ref[pl.ds(0, H), :],
                 preferred_element_type=jnp.float32)
         + jnp.dot(fg_ref[...], wl1_ref[pl.ds(H, EXTRA), :],
                   preferred_element_type=jnp.float32)
         + bl1_ref[...])
    t = _selu(t)
    t = _selu(jnp.dot(t, wl2_ref[...], preferred_element_type=jnp.float32)
              + bl2_ref[...])
    out_ref[...] = (jnp.dot(t, wl3_ref[...],
                            preferred_element_type=jnp.float32)
                    + bl3_ref[...])


def kernel(edge_index, feats_node, feats_graph, W1, b1, W2, b2, W3, b3,
           Wl1, bl1, Wl2, bl2, Wl3, bl3):
    f32 = jnp.float32
    pad_idx = N + jnp.arange(EPAD - E, dtype=jnp.int32) % (NPAD - N)
    src = jnp.concatenate([edge_index[0], pad_idx]).reshape(NROW, CH)
    dst = jnp.concatenate([edge_index[1], pad_idx]).reshape(NROW, CH)
    x_pad = jnp.concatenate(
        [feats_node, jnp.zeros((NPAD - N, D), f32)], axis=0)
    zeros2d = jnp.zeros((NPAD, D), f32)

    dg0, dg1 = _deg_kernel(src, dst)
    dg0 = dg0.reshape(2 * NPAD, 1)
    dg1 = dg1.reshape(2 * NPAD, 1)

    colv = lambda off: pl.BlockSpec((RB, 1), lambda i, off=off: (i + off, 0))
    col2 = pl.BlockSpec((RB, 2), lambda i: (i, 0))
    rowsD = pl.BlockSpec((RB, D), lambda i: (i, 0))
    whole = lambda s: pl.BlockSpec(s, lambda i: (0, 0))

    xs0, sc = pl.pallas_call(
        _prep_body,
        grid=(GB,),
        in_specs=[rowsD, colv(0), colv(0), colv(GB), colv(GB)],
        out_specs=[rowsD, col2],
        out_shape=[jax.ShapeDtypeStruct((NPAD, D), f32),
                   jax.ShapeDtypeStruct((NPAD, 2), f32)],
    )(x_pad, dg0, dg1, dg0, dg1)

    layer = pl.pallas_call(
        _layer_body,
        grid=(GB,),
        in_specs=[rowsD, rowsD, col2,
                  whole((D, H)), whole((1, H))],
        out_specs=rowsD,
        out_shape=jax.ShapeDtypeStruct((NPAD, D), f32),
    )

    x = xs0
    for W, b in ((W1, b1), (W2, b2)):
        pa, pb = _agg_kernel(src, dst, x, zeros2d)
        x = layer(pa, pb, sc, W, b.reshape(1, H))

    pa, pb = _agg_kernel(src, dst, x, zeros2d)
    out = pl.pallas_call(
        _final_body,
        out_shape=jax.ShapeDtypeStruct((1, 1), f32),
    )(pa, pb, sc, W3, b3.reshape(1, H), feats_graph, Wl1,
      bl1.reshape(1, 2 * H), Wl2, bl2.reshape(1, H), Wl3, bl3.reshape(1, 1))
    return out.reshape(-1)
